# speedup vs baseline: 7.1341x; 7.1341x over previous
import jax
import jax.numpy as jnp
from jax.experimental import pallas as pl

N, V, C_IN, C_OUT, T_IN, T_OUT = 16, 25, 64, 64, 10, 10
DIM, KS, E_PER = 3, 5, 48
KK = KS ** DIM

_HI = jax.lax.Precision.HIGHEST


def _elu(x):
    return jnp.where(x > 0, x, jnp.exp(jnp.minimum(x, 0.0)) - 1.0)


def _build_wblocks_kernel(ei_ref, ea_ref, wflat_ref, root_ref, out_ref):
    ea = ea_ref[:E_PER, :]
    v = jnp.clip(ea, 0.0, 1.0) * (KS - 1)
    v = jnp.minimum(v, KS - 1 - 1e-6)
    lo_f = jnp.floor(v)
    fr = v - lo_f
    lo = lo_f.astype(jnp.int32)

    kio = jax.lax.broadcasted_iota(jnp.int32, (E_PER, KK), 1)
    P = jnp.zeros((E_PER, KK), dtype=jnp.float32)
    for s in range(2 ** DIM):
        basis = jnp.ones((E_PER, 1), dtype=jnp.float32)
        widx = jnp.zeros((E_PER, 1), dtype=jnp.int32)
        off = 1
        for d in range(DIM):
            bit = (s >> d) & 1
            basis = basis * (fr[:, d:d + 1] if bit else (1.0 - fr[:, d:d + 1]))
            widx = widx + (lo[:, d:d + 1] + bit) * off
            off *= KS
        P = P + jnp.where(widx == kio, basis, 0.0)

    src = ei_ref[0:1, :E_PER]
    tgt = ei_ref[1:2, :E_PER]
    pvec = src * V + tgt
    pio = jax.lax.broadcasted_iota(jnp.int32, (V * V, E_PER), 0)
    Q = jnp.where(pio == pvec, 1.0, 0.0)

    M = jax.lax.dot_general(P, wflat_ref[...], (((1,), (0,)), ((), ())),
                            precision=_HI, preferred_element_type=jnp.float32)
    Wb = jax.lax.dot_general(Q, M, (((1,), (0,)), ((), ())),
                             precision=_HI, preferred_element_type=jnp.float32)
    rowio = jax.lax.broadcasted_iota(jnp.int32, (V * V, 1), 0)
    diag = jnp.where(rowio % (V + 1) == 0, 1.0, 0.0)
    out_ref[...] = Wb + diag * root_ref[...]


def _dense_pipeline_kernel(xb_ref, wbig_ref, rbd_ref, tbd_ref,
                           b1_ref, b2_ref, b3_ref, out_ref):
    xb = xb_ref[...]
    h1 = _elu(jax.lax.dot_general(xb, wbig_ref[...], (((1,), (0,)), ((), ())),
                                  precision=_HI,
                                  preferred_element_type=jnp.float32)
              + b1_ref[...])
    r = _elu(jax.lax.dot_general(xb, rbd_ref[...], (((1,), (0,)), ((), ())),
                                 precision=_HI,
                                 preferred_element_type=jnp.float32)
             + b2_ref[...])
    h2 = _elu(h1 + r)
    out = jax.lax.dot_general(tbd_ref[...], h2, (((1,), (0,)), ((), ())),
                              precision=_HI,
                              preferred_element_type=jnp.float32)
    out_ref[...] = _elu(out + b3_ref[...])


@jax.jit
def kernel(x, edge_index, edge_attr, W_spline, root_w, bias_spline,
           res_w, res_b, tcn_w, tcn_b):
    ei = edge_index.astype(jnp.int32)
    wflat = W_spline.reshape(KK, C_IN * C_OUT)
    root_row = root_w.reshape(1, C_IN * C_OUT)

    w_blocks = pl.pallas_call(
        _build_wblocks_kernel,
        out_shape=jax.ShapeDtypeStruct((V * V, C_IN * C_OUT), jnp.float32),
    )(ei, edge_attr, wflat, root_row)

    w_big = w_blocks.reshape(V, V, C_IN, C_OUT).transpose(0, 2, 1, 3)
    w_big = w_big.reshape(V * C_IN, V * C_OUT)

    xb = x.transpose(0, 3, 1, 2).reshape(N * T_IN, V * C_IN)

    rbd = jnp.kron(jnp.eye(V, dtype=jnp.float32), res_w.T)
    tbd = jnp.kron(jnp.eye(N, dtype=jnp.float32), tcn_w)
    b1 = jnp.tile(bias_spline, V)[None, :]
    b2 = jnp.tile(res_b, V)[None, :]
    b3 = jnp.tile(tcn_b, N)[:, None]

    out = pl.pallas_call(
        _dense_pipeline_kernel,
        out_shape=jax.ShapeDtypeStruct((N * T_OUT, V * C_OUT), jnp.float32),
    )(xb, w_big, rbd, tbd, b1, b2, b3)

    return out.reshape(N, T_OUT, V, C_OUT).transpose(0, 2, 3, 1)


# speedup vs baseline: 8.2175x; 1.1519x over previous
import jax
import jax.numpy as jnp
from jax.experimental import pallas as pl

N, V, C_IN, C_OUT, T_IN, T_OUT = 16, 25, 64, 64, 10, 10
DIM, KS, E_PER = 3, 5, 48
KK = KS ** DIM

_HI = jax.lax.Precision.DEFAULT


def _elu(x):
    return jnp.where(x > 0, x, jnp.exp(jnp.minimum(x, 0.0)) - 1.0)


def _build_wblocks_kernel(ei_ref, ea_ref, wflat_ref, root_ref, out_ref):
    ea = ea_ref[:E_PER, :]
    v = jnp.clip(ea, 0.0, 1.0) * (KS - 1)
    v = jnp.minimum(v, KS - 1 - 1e-6)
    lo_f = jnp.floor(v)
    fr = v - lo_f
    lo = lo_f.astype(jnp.int32)

    kio = jax.lax.broadcasted_iota(jnp.int32, (E_PER, KK), 1)
    P = jnp.zeros((E_PER, KK), dtype=jnp.float32)
    for s in range(2 ** DIM):
        basis = jnp.ones((E_PER, 1), dtype=jnp.float32)
        widx = jnp.zeros((E_PER, 1), dtype=jnp.int32)
        off = 1
        for d in range(DIM):
            bit = (s >> d) & 1
            basis = basis * (fr[:, d:d + 1] if bit else (1.0 - fr[:, d:d + 1]))
            widx = widx + (lo[:, d:d + 1] + bit) * off
            off *= KS
        P = P + jnp.where(widx == kio, basis, 0.0)

    src = ei_ref[0:1, :E_PER]
    tgt = ei_ref[1:2, :E_PER]
    pvec = src * V + tgt
    pio = jax.lax.broadcasted_iota(jnp.int32, (V * V, E_PER), 0)
    Q = jnp.where(pio == pvec, 1.0, 0.0)

    M = jax.lax.dot_general(P, wflat_ref[...], (((1,), (0,)), ((), ())),
                            precision=_HI, preferred_element_type=jnp.float32)
    Wb = jax.lax.dot_general(Q, M, (((1,), (0,)), ((), ())),
                             precision=_HI, preferred_element_type=jnp.float32)
    rowio = jax.lax.broadcasted_iota(jnp.int32, (V * V, 1), 0)
    diag = jnp.where(rowio % (V + 1) == 0, 1.0, 0.0)
    out_ref[...] = Wb + diag * root_ref[...]


def _dense_pipeline_kernel(xb_ref, wbig_ref, rbd_ref, tbd_ref,
                           b1_ref, b2_ref, b3_ref, out_ref):
    xb = xb_ref[...]
    h1 = _elu(jax.lax.dot_general(xb, wbig_ref[...], (((1,), (0,)), ((), ())),
                                  precision=_HI,
                                  preferred_element_type=jnp.float32)
              + b1_ref[...])
    r = _elu(jax.lax.dot_general(xb, rbd_ref[...], (((1,), (0,)), ((), ())),
                                 precision=_HI,
                                 preferred_element_type=jnp.float32)
             + b2_ref[...])
    h2 = _elu(h1 + r)
    out = jax.lax.dot_general(tbd_ref[...], h2, (((1,), (0,)), ((), ())),
                              precision=_HI,
                              preferred_element_type=jnp.float32)
    out_ref[...] = _elu(out + b3_ref[...])


@jax.jit
def kernel(x, edge_index, edge_attr, W_spline, root_w, bias_spline,
           res_w, res_b, tcn_w, tcn_b):
    ei = edge_index.astype(jnp.int32)
    wflat = W_spline.reshape(KK, C_IN * C_OUT)
    root_row = root_w.reshape(1, C_IN * C_OUT)

    w_blocks = pl.pallas_call(
        _build_wblocks_kernel,
        out_shape=jax.ShapeDtypeStruct((V * V, C_IN * C_OUT), jnp.float32),
    )(ei, edge_attr, wflat, root_row)

    w_big = w_blocks.reshape(V, V, C_IN, C_OUT).transpose(0, 2, 1, 3)
    w_big = w_big.reshape(V * C_IN, V * C_OUT)

    xb = x.transpose(0, 3, 1, 2).reshape(N * T_IN, V * C_IN)

    rbd = jnp.kron(jnp.eye(V, dtype=jnp.float32), res_w.T)
    tbd = jnp.kron(jnp.eye(N, dtype=jnp.float32), tcn_w)
    b1 = jnp.tile(bias_spline, V)[None, :]
    b2 = jnp.tile(res_b, V)[None, :]
    b3 = jnp.tile(tcn_b, N)[:, None]

    out = pl.pallas_call(
        _dense_pipeline_kernel,
        out_shape=jax.ShapeDtypeStruct((N * T_OUT, V * C_OUT), jnp.float32),
    )(xb, w_big, rbd, tbd, b1, b2, b3)

    return out.reshape(N, T_OUT, V, C_OUT).transpose(0, 2, 3, 1)


# speedup vs baseline: 11.0038x; 1.3391x over previous
import jax
import jax.numpy as jnp
from jax.experimental import pallas as pl

N, V, C_IN, C_OUT, T_IN, T_OUT = 16, 25, 64, 64, 10, 10
DIM, KS, E_PER = 3, 5, 48
KK = KS ** DIM

_HI = jax.lax.Precision.DEFAULT


def _elu(x):
    return jnp.where(x > 0, x, jnp.exp(jnp.minimum(x, 0.0)) - 1.0)


def _build_wblocks_kernel(ei_ref, ea_ref, wflat_ref, root_ref, out_ref):
    ea = ea_ref[:E_PER, :]
    v = jnp.clip(ea, 0.0, 1.0) * (KS - 1)
    v = jnp.minimum(v, KS - 1 - 1e-6)
    lo_f = jnp.floor(v)
    fr = v - lo_f
    lo = lo_f.astype(jnp.int32)

    kio = jax.lax.broadcasted_iota(jnp.int32, (E_PER, KK), 1)
    P = jnp.zeros((E_PER, KK), dtype=jnp.float32)
    for s in range(2 ** DIM):
        basis = jnp.ones((E_PER, 1), dtype=jnp.float32)
        widx = jnp.zeros((E_PER, 1), dtype=jnp.int32)
        off = 1
        for d in range(DIM):
            bit = (s >> d) & 1
            basis = basis * (fr[:, d:d + 1] if bit else (1.0 - fr[:, d:d + 1]))
            widx = widx + (lo[:, d:d + 1] + bit) * off
            off *= KS
        P = P + jnp.where(widx == kio, basis, 0.0)

    src = ei_ref[0:1, :E_PER]
    tgt = ei_ref[1:2, :E_PER]
    pvec = src * V + tgt
    pio = jax.lax.broadcasted_iota(jnp.int32, (V * V, E_PER), 0)
    Q = jnp.where(pio == pvec, 1.0, 0.0)

    M = jax.lax.dot_general(P, wflat_ref[...], (((1,), (0,)), ((), ())),
                            precision=_HI, preferred_element_type=jnp.float32)
    Wb = jax.lax.dot_general(Q, M, (((1,), (0,)), ((), ())),
                             precision=_HI, preferred_element_type=jnp.float32)
    rowio = jax.lax.broadcasted_iota(jnp.int32, (V * V, 1), 0)
    diag = jnp.where(rowio % (V + 1) == 0, 1.0, 0.0)
    out_ref[...] = Wb + diag * root_ref[...]


def _dense_pipeline_kernel(xb_ref, wbig_ref, reswt_ref, tbd_ref,
                           b1_ref, b2_ref, b3_ref, out_ref):
    xb = xb_ref[...]
    h1 = _elu(jax.lax.dot_general(xb, wbig_ref[...], (((1,), (0,)), ((), ())),
                                  precision=_HI,
                                  preferred_element_type=jnp.float32)
              + b1_ref[...])
    reswt = reswt_ref[...]
    parts = []
    for v in range(V):
        xv = xb[:, v * C_IN:(v + 1) * C_IN]
        parts.append(jax.lax.dot_general(
            xv, reswt, (((1,), (0,)), ((), ())),
            precision=_HI, preferred_element_type=jnp.float32))
    r = _elu(jnp.concatenate(parts, axis=1) + b2_ref[...])
    h2 = _elu(h1 + r)
    out = jax.lax.dot_general(tbd_ref[...], h2, (((1,), (0,)), ((), ())),
                              precision=_HI,
                              preferred_element_type=jnp.float32)
    out_ref[...] = _elu(out + b3_ref[...])


@jax.jit
def kernel(x, edge_index, edge_attr, W_spline, root_w, bias_spline,
           res_w, res_b, tcn_w, tcn_b):
    ei = edge_index.astype(jnp.int32)
    wflat = W_spline.reshape(KK, C_IN * C_OUT)
    root_row = root_w.reshape(1, C_IN * C_OUT)

    w_blocks = pl.pallas_call(
        _build_wblocks_kernel,
        out_shape=jax.ShapeDtypeStruct((V * V, C_IN * C_OUT), jnp.float32),
    )(ei, edge_attr, wflat, root_row)

    w_big = w_blocks.reshape(V, V, C_IN, C_OUT).transpose(0, 2, 1, 3)
    w_big = w_big.reshape(V * C_IN, V * C_OUT)

    xb = x.transpose(0, 3, 1, 2).reshape(N * T_IN, V * C_IN)

    tbd = jnp.kron(jnp.eye(N, dtype=jnp.float32), tcn_w)
    b1 = jnp.tile(bias_spline, V)[None, :]
    b2 = jnp.tile(res_b, V)[None, :]
    b3 = jnp.tile(tcn_b, N)[:, None]

    out = pl.pallas_call(
        _dense_pipeline_kernel,
        out_shape=jax.ShapeDtypeStruct((N * T_OUT, V * C_OUT), jnp.float32),
    )(xb, w_big, res_w.T, tbd, b1, b2, b3)

    return out.reshape(N, T_OUT, V, C_OUT).transpose(0, 2, 3, 1)


# speedup vs baseline: 24.7350x; 2.2479x over previous
import jax
import jax.numpy as jnp
from jax.experimental import pallas as pl

N, V, C_IN, C_OUT, T_IN, T_OUT = 16, 25, 64, 64, 10, 10
DIM, KS, E_PER = 3, 5, 48
KK = KS ** DIM

_HI = jax.lax.Precision.DEFAULT


def _elu(x):
    return jnp.where(x > 0, x, jnp.exp(jnp.minimum(x, 0.0)) - 1.0)


def _build_wblocks_kernel(ei_ref, ea_ref, wflat_ref, root_ref, out_ref):
    ea = ea_ref[:E_PER, :]
    v = jnp.clip(ea, 0.0, 1.0) * (KS - 1)
    v = jnp.minimum(v, KS - 1 - 1e-6)
    lo_f = jnp.floor(v)
    fr = v - lo_f
    lo = lo_f.astype(jnp.int32)

    kio = jax.lax.broadcasted_iota(jnp.int32, (E_PER, KK), 1)
    P = jnp.zeros((E_PER, KK), dtype=jnp.float32)
    for s in range(2 ** DIM):
        basis = jnp.ones((E_PER, 1), dtype=jnp.float32)
        widx = jnp.zeros((E_PER, 1), dtype=jnp.int32)
        off = 1
        for d in range(DIM):
            bit = (s >> d) & 1
            basis = basis * (fr[:, d:d + 1] if bit else (1.0 - fr[:, d:d + 1]))
            widx = widx + (lo[:, d:d + 1] + bit) * off
            off *= KS
        P = P + jnp.where(widx == kio, basis, 0.0)

    src = ei_ref[0:1, :E_PER]
    tgt = ei_ref[1:2, :E_PER]
    pvec = src * V + tgt
    pio = jax.lax.broadcasted_iota(jnp.int32, (V * V, E_PER), 0)
    Q = jnp.where(pio == pvec, 1.0, 0.0)

    M = jax.lax.dot_general(P, wflat_ref[...], (((1,), (0,)), ((), ())),
                            precision=_HI, preferred_element_type=jnp.float32)
    Wb = jax.lax.dot_general(Q, M, (((1,), (0,)), ((), ())),
                             precision=_HI, preferred_element_type=jnp.float32)
    rowio = jax.lax.broadcasted_iota(jnp.int32, (V * V, 1), 0)
    diag = jnp.where(rowio % (V + 1) == 0, 1.0, 0.0)
    Wb = Wb + diag * root_ref[...]
    w4 = Wb.reshape(V, V, C_IN, C_OUT).transpose(0, 2, 1, 3)
    out_ref[...] = w4.reshape(V * C_IN, V * C_OUT)


def _dense_pipeline_kernel(xb_ref, wbig_ref, reswt_ref, tbd_ref,
                           b1_ref, b2_ref, b3_ref, out_ref):
    xb = xb_ref[...]
    h1 = _elu(jax.lax.dot_general(xb, wbig_ref[...], (((1,), (0,)), ((), ())),
                                  precision=_HI,
                                  preferred_element_type=jnp.float32)
              + b1_ref[...])
    reswt = reswt_ref[...]
    parts = []
    for v in range(V):
        xv = xb[:, v * C_IN:(v + 1) * C_IN]
        parts.append(jax.lax.dot_general(
            xv, reswt, (((1,), (0,)), ((), ())),
            precision=_HI, preferred_element_type=jnp.float32))
    r = _elu(jnp.concatenate(parts, axis=1) + b2_ref[...])
    h2 = _elu(h1 + r)
    out = jax.lax.dot_general(tbd_ref[...], h2, (((1,), (0,)), ((), ())),
                              precision=_HI,
                              preferred_element_type=jnp.float32)
    out_ref[...] = _elu(out + b3_ref[...])


@jax.jit
def kernel(x, edge_index, edge_attr, W_spline, root_w, bias_spline,
           res_w, res_b, tcn_w, tcn_b):
    ei = edge_index.astype(jnp.int32)
    wflat = W_spline.reshape(KK, C_IN * C_OUT)
    root_row = root_w.reshape(1, C_IN * C_OUT)

    w_big = pl.pallas_call(
        _build_wblocks_kernel,
        out_shape=jax.ShapeDtypeStruct((V * C_IN, V * C_OUT), jnp.float32),
    )(ei, edge_attr, wflat, root_row)

    xb = x.transpose(0, 3, 1, 2).reshape(N * T_IN, V * C_IN)

    tbd = jnp.kron(jnp.eye(N, dtype=jnp.float32), tcn_w)
    b1 = jnp.tile(bias_spline, V)[None, :]
    b2 = jnp.tile(res_b, V)[None, :]
    b3 = jnp.tile(tcn_b, N)[:, None]

    out = pl.pallas_call(
        _dense_pipeline_kernel,
        out_shape=jax.ShapeDtypeStruct((N * T_OUT, V * C_OUT), jnp.float32),
    )(xb, w_big, res_w.T, tbd, b1, b2, b3)

    return out.reshape(N, T_OUT, V, C_OUT).transpose(0, 2, 3, 1)


# speedup vs baseline: 30.5106x; 1.2335x over previous
import jax
import jax.numpy as jnp
from jax.experimental import pallas as pl

N, V, C_IN, C_OUT, T_IN, T_OUT = 16, 25, 64, 64, 10, 10
DIM, KS, E_PER = 3, 5, 48
KK = KS ** DIM

_HI = jax.lax.Precision.DEFAULT


def _elu(x):
    return jnp.where(x > 0, x, jnp.exp(jnp.minimum(x, 0.0)) - 1.0)


def _fused_kernel(ei_ref, ea_ref, wflat_ref, root_ref, xb_ref, reswt_ref,
                  tbd_ref, b1_ref, b2_ref, b3_ref, out_ref):
    ea = ea_ref[:E_PER, :]
    v = jnp.clip(ea, 0.0, 1.0) * (KS - 1)
    v = jnp.minimum(v, KS - 1 - 1e-6)
    lo_f = jnp.floor(v)
    fr = v - lo_f
    lo = lo_f.astype(jnp.int32)

    kio = jax.lax.broadcasted_iota(jnp.int32, (E_PER, KK), 1)
    P = jnp.zeros((E_PER, KK), dtype=jnp.float32)
    for s in range(2 ** DIM):
        basis = jnp.ones((E_PER, 1), dtype=jnp.float32)
        widx = jnp.zeros((E_PER, 1), dtype=jnp.int32)
        off = 1
        for d in range(DIM):
            bit = (s >> d) & 1
            basis = basis * (fr[:, d:d + 1] if bit else (1.0 - fr[:, d:d + 1]))
            widx = widx + (lo[:, d:d + 1] + bit) * off
            off *= KS
        P = P + jnp.where(widx == kio, basis, 0.0)

    src = ei_ref[0:1, :E_PER]
    tgt = ei_ref[1:2, :E_PER]
    pvec = src * V + tgt
    pio = jax.lax.broadcasted_iota(jnp.int32, (V * V, E_PER), 0)
    Q = jnp.where(pio == pvec, 1.0, 0.0)

    M = jax.lax.dot_general(P, wflat_ref[...], (((1,), (0,)), ((), ())),
                            precision=_HI, preferred_element_type=jnp.float32)
    Wb = jax.lax.dot_general(Q, M, (((1,), (0,)), ((), ())),
                             precision=_HI, preferred_element_type=jnp.float32)
    rowio = jax.lax.broadcasted_iota(jnp.int32, (V * V, 1), 0)
    diag = jnp.where(rowio % (V + 1) == 0, 1.0, 0.0)
    Wb = Wb + diag * root_ref[...]
    w_big = Wb.reshape(V, V, C_IN, C_OUT).transpose(0, 2, 1, 3)
    w_big = w_big.reshape(V * C_IN, V * C_OUT)

    xb = xb_ref[...]
    h1 = _elu(jax.lax.dot_general(xb, w_big, (((1,), (0,)), ((), ())),
                                  precision=_HI,
                                  preferred_element_type=jnp.float32)
              + b1_ref[...])
    reswt = reswt_ref[...]
    parts = []
    for vv in range(V):
        xv = xb[:, vv * C_IN:(vv + 1) * C_IN]
        parts.append(jax.lax.dot_general(
            xv, reswt, (((1,), (0,)), ((), ())),
            precision=_HI, preferred_element_type=jnp.float32))
    r = _elu(jnp.concatenate(parts, axis=1) + b2_ref[...])
    h2 = _elu(h1 + r)
    out = jax.lax.dot_general(tbd_ref[...], h2, (((1,), (0,)), ((), ())),
                              precision=_HI,
                              preferred_element_type=jnp.float32)
    out_ref[...] = _elu(out + b3_ref[...])


@jax.jit
def kernel(x, edge_index, edge_attr, W_spline, root_w, bias_spline,
           res_w, res_b, tcn_w, tcn_b):
    ei = edge_index.astype(jnp.int32)
    wflat = W_spline.reshape(KK, C_IN * C_OUT)
    root_row = root_w.reshape(1, C_IN * C_OUT)

    xb = x.transpose(0, 3, 1, 2).reshape(N * T_IN, V * C_IN)

    tbd = jnp.kron(jnp.eye(N, dtype=jnp.float32), tcn_w)
    b1 = jnp.tile(bias_spline, V)[None, :]
    b2 = jnp.tile(res_b, V)[None, :]
    b3 = jnp.tile(tcn_b, N)[:, None]

    out = pl.pallas_call(
        _fused_kernel,
        out_shape=jax.ShapeDtypeStruct((N * T_OUT, V * C_OUT), jnp.float32),
    )(ei, edge_attr, wflat, root_row, xb, res_w.T, tbd, b1, b2, b3)

    return out.reshape(N, T_OUT, V, C_OUT).transpose(0, 2, 3, 1)
